# baseline (speedup 1.0000x reference)
import functools

import jax
import jax.numpy as jnp
from jax import lax
from jax.experimental import pallas as pl
from jax.experimental.pallas import tpu as pltpu
from jax.experimental.pallas import tpu_sc as plsc

V = 100000
H = 768
S = 512
B = 32
N_TOK = B * S
N_ELEM = float(N_TOK * H)
LANES = 16
G = H // LANES


def _sc_gather_kernel():
    info = plsc.get_sparse_core_info()
    nc, ns = info.num_cores, info.num_subcores
    nw = nc * ns
    sw = S // nw
    mesh = plsc.VectorSubcoreMesh(core_axis_name="c", subcore_axis_name="s")

    @functools.partial(
        pl.kernel,
        out_type=(
            jax.ShapeDtypeStruct((N_TOK, H), jnp.float32),
            jax.ShapeDtypeStruct((2 * nw * LANES,), jnp.float32),
        ),
        mesh=mesh,
        scratch_types=[
            pltpu.VMEM((2 * sw, H), jnp.float32),
            pltpu.VMEM((2, H), jnp.float32),
            pltpu.VMEM((B * sw,), jnp.int32),
            pltpu.VMEM((B * sw,), jnp.int32),
            pltpu.VMEM((sw, H), jnp.float32),
            pltpu.VMEM((2, LANES), jnp.float32),
            pltpu.SemaphoreType.DMA,
        ],
    )
    def sc_kernel(ids_hbm, tt_hbm, word_hbm, type_hbm, pos_hbm,
                  emb_hbm, part_hbm,
                  ptab, tybuf, idsbuf, ttbuf, gbuf, pbuf, gsem):
        wid = lax.axis_index("s") * nc + lax.axis_index("c")
        s0 = wid * sw
        tpw = B * sw

        pltpu.sync_copy(pos_hbm.at[pl.ds(s0, sw), :], ptab.at[pl.ds(0, sw), :])
        pltpu.sync_copy(pos_hbm.at[pl.ds(s0, sw), :], ptab.at[pl.ds(sw, sw), :])
        pltpu.sync_copy(type_hbm, tybuf)
        pltpu.sync_copy(ids_hbm.at[pl.ds(wid * tpw, tpw)], idsbuf)
        pltpu.sync_copy(tt_hbm.at[pl.ds(wid * tpw, tpw)], ttbuf)

        def build_row(i, t):
            def build_grp(j, _):
                sl = pl.ds(j * LANES, LANES)
                ptab[t * sw + i, sl] = ptab[t * sw + i, sl] + tybuf[t, sl]
                return 0
            lax.fori_loop(0, G, build_grp, 0)
            return 0
        lax.fori_loop(0, sw, lambda i, _: build_row(i, 0), 0)
        lax.fori_loop(0, sw, lambda i, _: build_row(i, 1), 0)

        zeros = jnp.zeros((LANES,), jnp.float32)

        def chunk(b, carry):
            sum_v, sq_v = carry
            pltpu.async_copy(
                word_hbm.at[idsbuf.at[pl.ds(b * sw, sw)]], gbuf, gsem
            ).wait()

            ttv = ttbuf[pl.ds(b * sw, LANES)]
            carry2 = (sum_v, sq_v)
            for i in range(sw):
                m = ttv[i] > 0

                def grp(j, c3, i=i, m=m):
                    s_v, q_v = c3
                    sl = pl.ds(j * LANES, LANES)
                    pt = jnp.where(m, ptab[sw + i, sl], ptab[i, sl])
                    x = gbuf[i, sl] + pt
                    gbuf[i, sl] = x
                    return (s_v + x, q_v + x * x)

                carry2 = lax.fori_loop(0, G, grp, carry2)
            sum_v, sq_v = carry2
            base = b * S + s0
            pltpu.sync_copy(gbuf, emb_hbm.at[pl.ds(base, sw), :])
            return (sum_v, sq_v)

        sum_v, sq_v = lax.fori_loop(0, B, chunk, (zeros, zeros))
        pbuf[0, :] = sum_v
        pbuf[1, :] = sq_v
        pltpu.sync_copy(pbuf.at[0], part_hbm.at[pl.ds(wid * LANES, LANES)])
        pltpu.sync_copy(pbuf.at[1], part_hbm.at[pl.ds(nw * LANES + wid * LANES, LANES)])

    return sc_kernel


def _tc_normalize(part, emb):
    blk = 1024

    half = part.shape[0] // 2

    def body(p_ref, x_ref, o_ref):
        p = p_ref[...]
        s = jnp.sum(p[:half])
        q = jnp.sum(p[half:])
        mean = s / N_ELEM
        var = q / N_ELEM - mean * mean
        inv = lax.rsqrt(var + 1e-5)
        o_ref[...] = (x_ref[...] - mean) * inv

    return pl.pallas_call(
        body,
        grid=(N_TOK // blk,),
        in_specs=[
            pl.BlockSpec(part.shape, lambda i: (0,)),
            pl.BlockSpec((blk, H), lambda i: (i, 0)),
        ],
        out_specs=pl.BlockSpec((blk, H), lambda i: (i, 0)),
        out_shape=jax.ShapeDtypeStruct((N_TOK, H), jnp.float32),
    )(part, emb)


def kernel(input_ids, token_type_ids, word_emb, type_emb, pos_emb):
    info = plsc.get_sparse_core_info()
    nw = info.num_cores * info.num_subcores
    sw = S // nw
    ids = input_ids.astype(jnp.int32).reshape(B, nw, sw).transpose(1, 0, 2).reshape(-1)
    tt = token_type_ids.astype(jnp.int32).reshape(B, nw, sw).transpose(1, 0, 2).reshape(-1)
    emb, part = _sc_gather_kernel()(ids, tt, word_emb, type_emb, pos_emb)
    out = _tc_normalize(part, emb)
    return out.reshape(B, S, H)


# speedup vs baseline: 1.4262x; 1.4262x over previous
import functools

import jax
import jax.numpy as jnp
from jax import lax
from jax.experimental import pallas as pl
from jax.experimental.pallas import tpu as pltpu
from jax.experimental.pallas import tpu_sc as plsc

V = 100000
H = 768
S = 512
B = 32
N_TOK = B * S
N_ELEM = float(N_TOK * H)
LANES = 16
G = H // LANES


def _sc_gather_kernel():
    info = plsc.get_sparse_core_info()
    nc, ns = info.num_cores, info.num_subcores
    nw = nc * ns
    sw = S // nw
    mesh = plsc.VectorSubcoreMesh(core_axis_name="c", subcore_axis_name="s")

    @functools.partial(
        pl.kernel,
        out_type=(
            jax.ShapeDtypeStruct((N_TOK, H), jnp.float32),
            jax.ShapeDtypeStruct((2 * nw * LANES,), jnp.float32),
        ),
        mesh=mesh,
        scratch_types=[
            pltpu.VMEM((2 * sw, H), jnp.float32),
            pltpu.VMEM((2, H), jnp.float32),
            pltpu.VMEM((B * sw,), jnp.int32),
            pltpu.VMEM((B * sw,), jnp.int32),
            pltpu.VMEM((2, sw, H), jnp.float32),
            pltpu.VMEM((2, sw, H), jnp.float32),
            pltpu.VMEM((2, LANES), jnp.float32),
            pltpu.SemaphoreType.DMA,
            pltpu.SemaphoreType.DMA,
            pltpu.SemaphoreType.DMA,
            pltpu.SemaphoreType.DMA,
        ],
    )
    def sc_kernel(ids_hbm, tt_hbm, word_hbm, type_hbm, pos_hbm,
                  emb_hbm, part_hbm,
                  ptab, tybuf, idsbuf, ttbuf, gbuf, obuf, pbuf,
                  gsem0, gsem1, wsem0, wsem1):
        wid = lax.axis_index("s") * nc + lax.axis_index("c")
        s0 = wid * sw
        tpw = B * sw
        gsems = (gsem0, gsem1)
        wsems = (wsem0, wsem1)

        pltpu.sync_copy(pos_hbm.at[pl.ds(s0, sw), :], ptab.at[pl.ds(0, sw), :])
        pltpu.sync_copy(pos_hbm.at[pl.ds(s0, sw), :], ptab.at[pl.ds(sw, sw), :])
        pltpu.sync_copy(type_hbm, tybuf)
        pltpu.sync_copy(ids_hbm.at[pl.ds(wid * tpw, tpw)], idsbuf)
        pltpu.sync_copy(tt_hbm.at[pl.ds(wid * tpw, tpw)], ttbuf)

        def start_gather(b, k):
            return pltpu.async_copy(
                word_hbm.at[idsbuf.at[pl.ds(b * sw, sw)]], gbuf.at[k], gsems[k]
            )

        start_gather(0, 0)
        start_gather(1, 1)

        def build_row(i, t):
            def build_grp(j, _):
                sl = pl.ds(j * LANES, LANES)
                ptab[t * sw + i, sl] = ptab[t * sw + i, sl] + tybuf[t, sl]
                return 0
            lax.fori_loop(0, G, build_grp, 0)
            return 0
        lax.fori_loop(0, sw, lambda i, _: build_row(i, 0), 0)
        lax.fori_loop(0, sw, lambda i, _: build_row(i, 1), 0)

        zeros = jnp.zeros((LANES,), jnp.float32)

        def pair(g, carry):
            sum_v, sq_v = carry
            for k in range(2):
                b = g * 2 + k
                pltpu.make_async_copy(
                    word_hbm.at[idsbuf.at[pl.ds(b * sw, sw)]], gbuf.at[k], gsems[k]
                ).wait()

                @pl.when(g > 0)
                def _drain(k=k, b=b):
                    prev = (b - 2) * S + s0
                    pltpu.make_async_copy(
                        obuf.at[k], emb_hbm.at[pl.ds(prev, sw), :], wsems[k]
                    ).wait()

                ttv = ttbuf[pl.ds(b * sw, LANES)]
                for i in range(sw):
                    row = jnp.where(ttv[i] > 0, sw + i, i)

                    def grp(j, c3, i=i, k=k, row=row):
                        s_v, q_v = c3
                        sl = pl.ds(j * LANES, LANES)
                        x = gbuf[k, i, sl] + ptab[row, sl]
                        obuf[k, i, sl] = x
                        return (s_v + x, q_v + x * x)

                    sum_v, sq_v = lax.fori_loop(0, G, grp, (sum_v, sq_v))

                @pl.when(g < B // 2 - 1)
                def _next(k=k, b=b):
                    start_gather(b + 2, k)

                pltpu.async_copy(
                    obuf.at[k], emb_hbm.at[pl.ds(b * S + s0, sw), :], wsems[k]
                )
            return (sum_v, sq_v)

        sum_v, sq_v = lax.fori_loop(0, B // 2, pair, (zeros, zeros))
        for k in range(2):
            b = B - 2 + k
            pltpu.make_async_copy(
                obuf.at[k], emb_hbm.at[pl.ds(b * S + s0, sw), :], wsems[k]
            ).wait()
        pbuf[0, :] = sum_v
        pbuf[1, :] = sq_v
        pltpu.sync_copy(pbuf.at[0], part_hbm.at[pl.ds(wid * LANES, LANES)])
        pltpu.sync_copy(pbuf.at[1], part_hbm.at[pl.ds(nw * LANES + wid * LANES, LANES)])

    return sc_kernel


def _tc_normalize(part, emb):
    blk = 1024

    half = part.shape[0] // 2

    def body(p_ref, x_ref, o_ref):
        p = p_ref[...]
        s = jnp.sum(p[:half])
        q = jnp.sum(p[half:])
        mean = s / N_ELEM
        var = q / N_ELEM - mean * mean
        inv = lax.rsqrt(var + 1e-5)
        o_ref[...] = (x_ref[...] - mean) * inv

    return pl.pallas_call(
        body,
        grid=(N_TOK // blk,),
        in_specs=[
            pl.BlockSpec(part.shape, lambda i: (0,)),
            pl.BlockSpec((blk, H), lambda i: (i, 0)),
        ],
        out_specs=pl.BlockSpec((blk, H), lambda i: (i, 0)),
        out_shape=jax.ShapeDtypeStruct((N_TOK, H), jnp.float32),
    )(part, emb)


def kernel(input_ids, token_type_ids, word_emb, type_emb, pos_emb):
    info = plsc.get_sparse_core_info()
    nw = info.num_cores * info.num_subcores
    sw = S // nw
    ids = input_ids.astype(jnp.int32).reshape(B, nw, sw).transpose(1, 0, 2).reshape(-1)
    tt = token_type_ids.astype(jnp.int32).reshape(B, nw, sw).transpose(1, 0, 2).reshape(-1)
    emb, part = _sc_gather_kernel()(ids, tt, word_emb, type_emb, pos_emb)
    out = _tc_normalize(part, emb)
    return out.reshape(B, S, H)


# speedup vs baseline: 1.5210x; 1.0665x over previous
import functools

import jax
import jax.numpy as jnp
from jax import lax
from jax.experimental import pallas as pl
from jax.experimental.pallas import tpu as pltpu
from jax.experimental.pallas import tpu_sc as plsc

V = 100000
H = 768
S = 512
B = 32
N_TOK = B * S
N_ELEM = float(N_TOK * H)


def _sc_gather_kernel():
    info = plsc.get_sparse_core_info()
    nc, ns = info.num_cores, info.num_subcores
    nw = nc * ns
    tpw = N_TOK // nw
    ch = 32
    nch = tpw // ch
    nbuf = 4
    mesh = plsc.VectorSubcoreMesh(core_axis_name="c", subcore_axis_name="s")

    @functools.partial(
        pl.kernel,
        out_type=jax.ShapeDtypeStruct((N_TOK, H), jnp.float32),
        mesh=mesh,
        scratch_types=[
            pltpu.VMEM((tpw,), jnp.int32),
            pltpu.VMEM((nbuf, ch, H), jnp.float32),
        ]
        + [pltpu.SemaphoreType.DMA] * (2 * nbuf),
    )
    def sc_kernel(ids_hbm, word_hbm, raw_hbm, idsbuf, rbuf, *sems):
        gsems, wsems = sems[:nbuf], sems[nbuf:]
        wid = lax.axis_index("s") * nc + lax.axis_index("c")
        base = wid * tpw
        pltpu.sync_copy(ids_hbm.at[pl.ds(base, tpw)], idsbuf)

        def start_gather(j):
            k = j % nbuf
            pltpu.async_copy(
                word_hbm.at[idsbuf.at[pl.ds(j * ch, ch)]], rbuf.at[k], gsems[k]
            )

        def wait_gather(j):
            k = j % nbuf
            pltpu.make_async_copy(
                word_hbm.at[idsbuf.at[pl.ds(j * ch, ch)]], rbuf.at[k], gsems[k]
            ).wait()

        def start_write(j):
            k = j % nbuf
            pltpu.async_copy(
                rbuf.at[k], raw_hbm.at[pl.ds(base + j * ch, ch), :], wsems[k]
            )

        def wait_write(j):
            k = j % nbuf
            pltpu.make_async_copy(
                rbuf.at[k], raw_hbm.at[pl.ds(base + j * ch, ch), :], wsems[k]
            ).wait()

        start_gather(0)
        start_gather(1)
        for j in range(nch):
            wait_gather(j)
            start_write(j)
            if j + 2 < nch:
                if j >= 2:
                    wait_write(j - 2)
                start_gather(j + 2)
        for j in range(nch - nbuf, nch):
            wait_write(j)

    return sc_kernel


def _tc_norm(raw, ttmat, pos, ty2):
    blk = 512
    nb = N_TOK // blk

    def body(raw_ref, tt_ref, pos_ref, ty_ref, o_ref, acc):
        p = pl.program_id(0)
        i = pl.program_id(1)
        x = (
            raw_ref[...]
            + pos_ref[...]
            + jnp.dot(tt_ref[...], ty_ref[...],
                      precision=lax.Precision.HIGHEST,
                      preferred_element_type=jnp.float32)
        )

        @pl.when((p == 0) & (i == 0))
        def _init():
            acc[0] = 0.0
            acc[1] = 0.0

        @pl.when(p == 0)
        def _accum():
            acc[0] += jnp.sum(x)
            acc[1] += jnp.sum(x * x)

        @pl.when(p == 1)
        def _norm():
            mean = acc[0] / N_ELEM
            var = acc[1] / N_ELEM - mean * mean
            o_ref[...] = (x - mean) * lax.rsqrt(var + 1e-5)

    return pl.pallas_call(
        body,
        grid=(2, nb),
        in_specs=[
            pl.BlockSpec((blk, H), lambda p, i: (i, 0)),
            pl.BlockSpec((blk, 2), lambda p, i: (i, 0)),
            pl.BlockSpec((S, H), lambda p, i: (0, 0)),
            pl.BlockSpec((2, H), lambda p, i: (0, 0)),
        ],
        out_specs=pl.BlockSpec((blk, H), lambda p, i: (i * p, 0)),
        out_shape=jax.ShapeDtypeStruct((N_TOK, H), jnp.float32),
        scratch_shapes=[pltpu.SMEM((2,), jnp.float32)],
    )(raw, ttmat, pos, ty2)


def kernel(input_ids, token_type_ids, word_emb, type_emb, pos_emb):
    ids = input_ids.reshape(-1).astype(jnp.int32)
    raw = _sc_gather_kernel()(ids, word_emb)
    ttf = token_type_ids.reshape(-1).astype(jnp.float32)
    ttmat = jnp.stack([jnp.ones_like(ttf), ttf], axis=-1)
    ty2 = jnp.stack([type_emb[0], type_emb[1] - type_emb[0]])
    out = _tc_norm(raw, ttmat, pos_emb, ty2)
    return out.reshape(B, S, H)


# speedup vs baseline: 1.9528x; 1.2840x over previous
import functools

import jax
import jax.numpy as jnp
from jax import lax
from jax.experimental import pallas as pl
from jax.experimental.pallas import tpu as pltpu
from jax.experimental.pallas import tpu_sc as plsc

V = 100000
H = 768
S = 512
B = 32
N_TOK = B * S
N_ELEM = float(N_TOK * H)


def _sc_gather_kernel():
    info = plsc.get_sparse_core_info()
    nc, ns = info.num_cores, info.num_subcores
    nw = nc * ns
    tpw = N_TOK // nw
    ch = 32
    nch = tpw // ch
    nbuf = 4
    mesh = plsc.VectorSubcoreMesh(core_axis_name="c", subcore_axis_name="s")

    @functools.partial(
        pl.kernel,
        out_type=jax.ShapeDtypeStruct((N_TOK, H), jnp.float32),
        mesh=mesh,
        scratch_types=[
            pltpu.VMEM((tpw,), jnp.int32),
            pltpu.VMEM((nbuf, ch, H), jnp.float32),
        ]
        + [pltpu.SemaphoreType.DMA] * (2 * nbuf),
    )
    def sc_kernel(ids_hbm, word_hbm, raw_hbm, idsbuf, rbuf, *sems):
        gsems, wsems = sems[:nbuf], sems[nbuf:]
        wid = lax.axis_index("s") * nc + lax.axis_index("c")
        base = wid * tpw
        pltpu.sync_copy(ids_hbm.at[pl.ds(base, tpw)], idsbuf)

        def start_gather(j):
            k = j % nbuf
            pltpu.async_copy(
                word_hbm.at[idsbuf.at[pl.ds(j * ch, ch)]], rbuf.at[k], gsems[k]
            )

        def wait_gather(j):
            k = j % nbuf
            pltpu.make_async_copy(
                word_hbm.at[idsbuf.at[pl.ds(j * ch, ch)]], rbuf.at[k], gsems[k]
            ).wait()

        def start_write(j):
            k = j % nbuf
            pltpu.async_copy(
                rbuf.at[k], raw_hbm.at[pl.ds(base + j * ch, ch), :], wsems[k]
            )

        def wait_write(j):
            k = j % nbuf
            pltpu.make_async_copy(
                rbuf.at[k], raw_hbm.at[pl.ds(base + j * ch, ch), :], wsems[k]
            ).wait()

        start_gather(0)
        start_gather(1)
        for j in range(nch):
            wait_gather(j)
            start_write(j)
            if j + 2 < nch:
                if j >= 2:
                    wait_write(j - 2)
                start_gather(j + 2)
        for j in range(nch - nbuf, nch):
            wait_write(j)

    return sc_kernel


def _tc_norm(raw, ttmat, pos, ty2):
    blk = 512
    nb = N_TOK // blk

    def body(raw_ref, tt_ref, pos_ref, ty_ref, o_ref, acc, xbuf):
        p = pl.program_id(0)
        i = pl.program_id(1)

        @pl.when((p == 0) & (i == 0))
        def _init():
            acc[0] = 0.0
            acc[1] = 0.0

        @pl.when(p == 0)
        def _accum():
            x = (
                raw_ref[...]
                + pos_ref[...]
                + jnp.dot(tt_ref[...], ty_ref[...],
                          precision=lax.Precision.HIGHEST,
                          preferred_element_type=jnp.float32)
            )
            xbuf[pl.ds(i * blk, blk), :] = x
            acc[0] += jnp.sum(x)
            acc[1] += jnp.sum(x * x)

        @pl.when(p == 1)
        def _norm():
            mean = acc[0] / N_ELEM
            var = acc[1] / N_ELEM - mean * mean
            x = xbuf[pl.ds(i * blk, blk), :]
            o_ref[...] = (x - mean) * lax.rsqrt(var + 1e-5)

    return pl.pallas_call(
        body,
        grid=(2, nb),
        in_specs=[
            pl.BlockSpec((blk, H), lambda p, i: (i * (1 - p), 0)),
            pl.BlockSpec((blk, 2), lambda p, i: (i * (1 - p), 0)),
            pl.BlockSpec((S, H), lambda p, i: (0, 0)),
            pl.BlockSpec((2, H), lambda p, i: (0, 0)),
        ],
        out_specs=pl.BlockSpec((blk, H), lambda p, i: (i * p, 0)),
        out_shape=jax.ShapeDtypeStruct((N_TOK, H), jnp.float32),
        scratch_shapes=[
            pltpu.SMEM((2,), jnp.float32),
            pltpu.VMEM((N_TOK, H), jnp.float32),
        ],
    )(raw, ttmat, pos, ty2)


def kernel(input_ids, token_type_ids, word_emb, type_emb, pos_emb):
    ids = input_ids.reshape(-1).astype(jnp.int32)
    raw = _sc_gather_kernel()(ids, word_emb)
    ttf = token_type_ids.reshape(-1).astype(jnp.float32)
    ttmat = jnp.stack([jnp.ones_like(ttf), ttf], axis=-1)
    ty2 = jnp.stack([type_emb[0], type_emb[1] - type_emb[0]])
    out = _tc_norm(raw, ttmat, pos_emb, ty2)
    return out.reshape(B, S, H)


# speedup vs baseline: 1.9534x; 1.0003x over previous
import functools

import jax
import jax.numpy as jnp
from jax import lax
from jax.experimental import pallas as pl
from jax.experimental.pallas import tpu as pltpu
from jax.experimental.pallas import tpu_sc as plsc

V = 100000
H = 768
S = 512
B = 32
N_TOK = B * S
N_ELEM = float(N_TOK * H)


def _sc_gather_kernel():
    info = plsc.get_sparse_core_info()
    nc, ns = info.num_cores, info.num_subcores
    nw = nc * ns
    tpw = N_TOK // nw
    ch = 32
    nch = tpw // ch
    nbuf = 5
    mesh = plsc.VectorSubcoreMesh(core_axis_name="c", subcore_axis_name="s")

    @functools.partial(
        pl.kernel,
        out_type=jax.ShapeDtypeStruct((N_TOK, H), jnp.float32),
        mesh=mesh,
        scratch_types=[
            pltpu.VMEM((tpw,), jnp.int32),
            pltpu.VMEM((nbuf, ch, H), jnp.float32),
        ]
        + [pltpu.SemaphoreType.DMA] * (2 * nbuf),
    )
    def sc_kernel(ids_hbm, word_hbm, raw_hbm, idsbuf, rbuf, *sems):
        gsems, wsems = sems[:nbuf], sems[nbuf:]
        wid = lax.axis_index("s") * nc + lax.axis_index("c")
        base = wid * tpw
        pltpu.sync_copy(ids_hbm.at[pl.ds(base, tpw)], idsbuf)

        def start_gather(j):
            k = j % nbuf
            pltpu.async_copy(
                word_hbm.at[idsbuf.at[pl.ds(j * ch, ch)]], rbuf.at[k], gsems[k]
            )

        def wait_gather(j):
            k = j % nbuf
            pltpu.make_async_copy(
                word_hbm.at[idsbuf.at[pl.ds(j * ch, ch)]], rbuf.at[k], gsems[k]
            ).wait()

        def start_write(j):
            k = j % nbuf
            pltpu.async_copy(
                rbuf.at[k], raw_hbm.at[pl.ds(base + j * ch, ch), :], wsems[k]
            )

        def wait_write(j):
            k = j % nbuf
            pltpu.make_async_copy(
                rbuf.at[k], raw_hbm.at[pl.ds(base + j * ch, ch), :], wsems[k]
            ).wait()

        for j in range(min(nbuf, nch)):
            start_gather(j)
        for j in range(nch):
            wait_gather(j)
            start_write(j)
            if j + nbuf < nch:
                wait_write(j)
                start_gather(j + nbuf)
        for j in range(max(0, nch - nbuf), nch):
            wait_write(j)

    return sc_kernel


def _tc_norm(raw, ttmat, pos, ty2):
    blk = 512
    nb = N_TOK // blk

    def body(raw_ref, tt_ref, pos_ref, ty_ref, o_ref, acc, xbuf):
        p = pl.program_id(0)
        i = pl.program_id(1)

        @pl.when((p == 0) & (i == 0))
        def _init():
            acc[0] = 0.0
            acc[1] = 0.0

        @pl.when(p == 0)
        def _accum():
            x = (
                raw_ref[...]
                + pos_ref[...]
                + jnp.dot(tt_ref[...], ty_ref[...],
                          precision=lax.Precision.HIGHEST,
                          preferred_element_type=jnp.float32)
            )
            xbuf[pl.ds(i * blk, blk), :] = x
            acc[0] += jnp.sum(x)
            acc[1] += jnp.sum(x * x)

        @pl.when(p == 1)
        def _norm():
            mean = acc[0] / N_ELEM
            var = acc[1] / N_ELEM - mean * mean
            x = xbuf[pl.ds(i * blk, blk), :]
            o_ref[...] = (x - mean) * lax.rsqrt(var + 1e-5)

    return pl.pallas_call(
        body,
        grid=(2, nb),
        in_specs=[
            pl.BlockSpec((blk, H), lambda p, i: (i * (1 - p), 0)),
            pl.BlockSpec((blk, 2), lambda p, i: (i * (1 - p), 0)),
            pl.BlockSpec((S, H), lambda p, i: (0, 0)),
            pl.BlockSpec((2, H), lambda p, i: (0, 0)),
        ],
        out_specs=pl.BlockSpec((blk, H), lambda p, i: (i * p, 0)),
        out_shape=jax.ShapeDtypeStruct((N_TOK, H), jnp.float32),
        scratch_shapes=[
            pltpu.SMEM((2,), jnp.float32),
            pltpu.VMEM((N_TOK, H), jnp.float32),
        ],
    )(raw, ttmat, pos, ty2)


def kernel(input_ids, token_type_ids, word_emb, type_emb, pos_emb):
    ids = input_ids.reshape(-1).astype(jnp.int32)
    raw = _sc_gather_kernel()(ids, word_emb)
    ttf = token_type_ids.reshape(-1).astype(jnp.float32)
    ttmat = jnp.stack([jnp.ones_like(ttf), ttf], axis=-1)
    ty2 = jnp.stack([type_emb[0], type_emb[1] - type_emb[0]])
    out = _tc_norm(raw, ttmat, pos_emb, ty2)
    return out.reshape(B, S, H)
